# baseline (speedup 1.0000x reference)
import functools

import jax
import jax.numpy as jnp
from jax import lax
from jax.experimental import pallas as pl
from jax.experimental.pallas import tpu as pltpu
from jax.experimental.pallas import tpu_sc as plsc

N = 10000
D = 128
E = 320000

NC = 2
NS = 16
CH = 128
TOTCH = E // CH
CORECH = TOTCH // NC
CH0 = 80
CHR = 78
NACC = 10240

_SC_MESH = plsc.VectorSubcoreMesh(
    core_axis_name="c", subcore_axis_name="s", num_cores=NC, num_subcores=NS)


def _tile_chunks(c, s):
    base = c * CORECH + lax.select(s == 0, 0, CH0 + CHR * (s - 1))
    nch = lax.select(s == 0, CH0, CHR)
    return base, nch


@functools.partial(
    pl.kernel,
    out_type=jax.ShapeDtypeStruct((NC * N,), jnp.float32),
    mesh=_SC_MESH,
    scratch_types=[
        pltpu.VMEM((CH0 * CH,), jnp.int32),
        pltpu.VMEM((CH,), jnp.float32),
        pltpu.VMEM((1000,), jnp.float32),
        pltpu.VMEM_SHARED((NACC,), jnp.float32),
    ],
)
def _sc_degree(ef_hbm, ones_hbm, zeros_hbm, out_hbm, dst_v, ones_v,
               stage_v, acc_sh):
    c = lax.axis_index("c")
    s = lax.axis_index("s")
    base, nch = _tile_chunks(c, s)

    @pl.when(s < 10)
    def _zero():
        pltpu.sync_copy(zeros_hbm, stage_v)
        pltpu.sync_copy(stage_v, acc_sh.at[pl.ds(s * 1000, 1000)])

    @pl.when(s == 0)
    def _load0():
        pltpu.sync_copy(ef_hbm.at[pl.ds(E + base * CH, CH0 * CH)], dst_v)

    @pl.when(s > 0)
    def _loadr():
        pltpu.sync_copy(ef_hbm.at[pl.ds(E + base * CH, CHR * CH)],
                        dst_v.at[pl.ds(0, CHR * CH)])

    pltpu.sync_copy(ones_hbm, ones_v)
    plsc.subcore_barrier()

    def body(i, carry):
        pltpu.sync_copy(ones_v, acc_sh.at[dst_v.at[pl.ds(i * CH, CH)]],
                        add=True)
        return carry

    lax.fori_loop(0, nch, body, 0, unroll=False)
    plsc.subcore_barrier()

    @pl.when(s < 10)
    def _writeout():
        pltpu.sync_copy(acc_sh.at[pl.ds(s * 1000, 1000)], stage_v)
        pltpu.sync_copy(stage_v, out_hbm.at[pl.ds(c * N + s * 1000, 1000)])


@functools.partial(
    pl.kernel,
    out_type=jax.ShapeDtypeStruct((NC, N, D), jnp.float32),
    mesh=_SC_MESH,
    scratch_types=[
        pltpu.VMEM((CH,), jnp.int32),
        pltpu.VMEM((CH,), jnp.int32),
        pltpu.VMEM((CH,), jnp.int32),
        pltpu.VMEM((CH,), jnp.int32),
        pltpu.VMEM((CH, D), jnp.float32),
        pltpu.VMEM((CH, D), jnp.float32),
        pltpu.VMEM((40, D), jnp.float32),
        pltpu.VMEM_SHARED((NACC, D), jnp.float32),
        pltpu.SemaphoreType.DMA,
        pltpu.SemaphoreType.DMA,
    ],
)
def _sc_aggregate(g_hbm, ef_hbm, zeros_hbm, out_hbm,
                  ibs0, ibd0, ibs1, ibd1, rows0_v, rows1_v, stage_v, acc_sh,
                  sem0, sem1):
    c = lax.axis_index("c")
    s = lax.axis_index("s")
    base, nch = _tile_chunks(c, s)

    pltpu.sync_copy(zeros_hbm, stage_v)
    for j in range(16):
        pltpu.sync_copy(stage_v, acc_sh.at[pl.ds(s * 640 + j * 40, 40)])

    pltpu.sync_copy(ef_hbm.at[pl.ds(base * CH, CH)], ibs0)
    pltpu.sync_copy(ef_hbm.at[pl.ds(E + base * CH, CH)], ibd0)
    pltpu.sync_copy(ef_hbm.at[pl.ds((base + 1) * CH, CH)], ibs1)
    pltpu.sync_copy(ef_hbm.at[pl.ds(E + (base + 1) * CH, CH)], ibd1)
    plsc.subcore_barrier()

    pltpu.async_copy(g_hbm.at[ibs0.at[pl.ds(0, CH)]], rows0_v, sem0)

    def body(k, carry):
        i0 = 2 * k
        pltpu.async_copy(g_hbm.at[ibs1.at[pl.ds(0, CH)]], rows1_v, sem1)
        pltpu.make_async_copy(g_hbm.at[ibs0.at[pl.ds(0, CH)]], rows0_v, sem0).wait()
        pltpu.sync_copy(rows0_v, acc_sh.at[ibd0.at[pl.ds(0, CH)]], add=True)

        @pl.when(i0 + 2 < nch)
        def _next_even():
            pltpu.sync_copy(ef_hbm.at[pl.ds((base + i0 + 2) * CH, CH)], ibs0)
            pltpu.sync_copy(
                ef_hbm.at[pl.ds(E + (base + i0 + 2) * CH, CH)], ibd0)
            pltpu.async_copy(g_hbm.at[ibs0.at[pl.ds(0, CH)]], rows0_v, sem0)

        pltpu.make_async_copy(g_hbm.at[ibs1.at[pl.ds(0, CH)]], rows1_v, sem1).wait()
        pltpu.sync_copy(rows1_v, acc_sh.at[ibd1.at[pl.ds(0, CH)]], add=True)

        @pl.when(i0 + 3 < nch)
        def _next_odd():
            pltpu.sync_copy(ef_hbm.at[pl.ds((base + i0 + 3) * CH, CH)], ibs1)
            pltpu.sync_copy(
                ef_hbm.at[pl.ds(E + (base + i0 + 3) * CH, CH)], ibd1)

        return carry

    lax.fori_loop(0, nch // 2, body, 0, unroll=False)
    plsc.subcore_barrier()

    for j in range(16):
        blk = s + 16 * j

        @pl.when(blk < 250)
        def _wb():
            row = blk * 40
            pltpu.sync_copy(acc_sh.at[pl.ds(row, 40)], stage_v)
            pltpu.sync_copy(stage_v, out_hbm.at[c, pl.ds(row, 40)])


_BM = 2000


def _tc_scale_matmul_body(degA, degB, x_ref, w_ref, g_ref):
    dinv = lax.rsqrt(degA[...] + degB[...] + 1.0)
    h = jnp.dot(x_ref[...], w_ref[...], preferred_element_type=jnp.float32)
    g_ref[...] = h * dinv


def _tc_mid_body(degA, degB, aggA, aggB, g_ref, b_ref, w_ref, out_ref):
    dinv = lax.rsqrt(degA[...] + degB[...] + 1.0)
    h = (aggA[...] + aggB[...] + g_ref[...]) * dinv + b_ref[...]
    h = jnp.maximum(h, 0.0)
    out_ref[...] = jnp.dot(
        h, w_ref[...], preferred_element_type=jnp.float32) * dinv


def _tc_final_body(degA, degB, aggA, aggB, g_ref, b_ref, out_ref):
    dinv = lax.rsqrt(degA[...] + degB[...] + 1.0)
    h = (aggA[...] + aggB[...] + g_ref[...]) * dinv + b_ref[...]
    out_ref[...] = jnp.maximum(h, 0.0)


_col_spec = pl.BlockSpec((_BM, 1), lambda i: (i, 0))
_row_spec = pl.BlockSpec((_BM, D), lambda i: (i, 0))
_w_spec = pl.BlockSpec((D, D), lambda i: (0, 0))
_b_spec = pl.BlockSpec((1, D), lambda i: (0, 0))
_GRID = (N // _BM,)
_out_nd = jax.ShapeDtypeStruct((N, D), jnp.float32)

_tc_scale_matmul = pl.pallas_call(
    _tc_scale_matmul_body, grid=_GRID,
    in_specs=[_col_spec, _col_spec, _row_spec, _w_spec],
    out_specs=_row_spec, out_shape=_out_nd)

_tc_mid = pl.pallas_call(
    _tc_mid_body, grid=_GRID,
    in_specs=[_col_spec, _col_spec, _row_spec, _row_spec, _row_spec,
              _b_spec, _w_spec],
    out_specs=_row_spec, out_shape=_out_nd)

_tc_final = pl.pallas_call(
    _tc_final_body, grid=_GRID,
    in_specs=[_col_spec, _col_spec, _row_spec, _row_spec, _row_spec, _b_spec],
    out_specs=_row_spec, out_shape=_out_nd)


def kernel(x, edge_index, W1, b1, W2, b2):
    ef = edge_index.reshape(2 * E)
    zeros_n = jnp.zeros((1000,), jnp.float32)
    zeros_nd = jnp.zeros((40, D), jnp.float32)
    ones_ch = jnp.ones((CH,), jnp.float32)
    b1r = b1.reshape(1, D)
    b2r = b2.reshape(1, D)

    degp = _sc_degree(ef, ones_ch, zeros_n).reshape(NC, N)
    degA = degp[0][:, None]
    degB = degp[1][:, None]

    g1 = _tc_scale_matmul(degA, degB, x, W1)
    agg1 = _sc_aggregate(g1, ef, zeros_nd)
    g2 = _tc_mid(degA, degB, agg1[0], agg1[1], g1, b1r, W2)
    agg2 = _sc_aggregate(g2, ef, zeros_nd)
    out = _tc_final(degA, degB, agg2[0], agg2[1], g2, b2r)
    return out


# speedup vs baseline: 1.1777x; 1.1777x over previous
import functools

import jax
import jax.numpy as jnp
from jax import lax
from jax.experimental import pallas as pl
from jax.experimental.pallas import tpu as pltpu
from jax.experimental.pallas import tpu_sc as plsc

N = 10000
D = 128
E = 320000

NC = 2
NS = 16
CH = 128
TOTCH = E // CH
CORECH = TOTCH // NC
CH0 = 80
CHR = 78
NACC = 10240

_SC_MESH = plsc.VectorSubcoreMesh(
    core_axis_name="c", subcore_axis_name="s", num_cores=NC, num_subcores=NS)


def _tile_chunks(c, s):
    base = c * CORECH + lax.select(s == 0, 0, CH0 + CHR * (s - 1))
    nch = lax.select(s == 0, CH0, CHR)
    return base, nch


@functools.partial(
    pl.kernel,
    out_type=jax.ShapeDtypeStruct((NC * N,), jnp.float32),
    mesh=_SC_MESH,
    scratch_types=[
        pltpu.VMEM((2, CH0 * CH), jnp.int32),
        pltpu.VMEM((CH,), jnp.float32),
        pltpu.VMEM((1000,), jnp.float32),
        pltpu.VMEM_SHARED((NACC,), jnp.float32),
    ],
)
def _sc_degree(ef_hbm, ones_hbm, zeros_hbm, out_hbm, idx_v, ones_v,
               stage_v, acc_sh):
    c = lax.axis_index("c")
    s = lax.axis_index("s")
    base, nch = _tile_chunks(c, s)

    @pl.when(s < 10)
    def _zero():
        pltpu.sync_copy(zeros_hbm, stage_v)
        pltpu.sync_copy(stage_v, acc_sh.at[pl.ds(s * 1000, 1000)])

    @pl.when(s == 0)
    def _load0():
        pltpu.sync_copy(ef_hbm.at[:, pl.ds(base * CH, CH0 * CH)], idx_v)

    @pl.when(s > 0)
    def _loadr():
        pltpu.sync_copy(ef_hbm.at[:, pl.ds(base * CH, CHR * CH)],
                        idx_v.at[:, pl.ds(0, CHR * CH)])

    pltpu.sync_copy(ones_hbm, ones_v)
    plsc.subcore_barrier()

    def body(i, carry):
        pltpu.sync_copy(ones_v, acc_sh.at[idx_v.at[1, pl.ds(i * CH, CH)]],
                        add=True)
        return carry

    lax.fori_loop(0, nch, body, 0, unroll=False)
    plsc.subcore_barrier()

    @pl.when(s < 10)
    def _writeout():
        pltpu.sync_copy(acc_sh.at[pl.ds(s * 1000, 1000)], stage_v)
        pltpu.sync_copy(stage_v, out_hbm.at[pl.ds(c * N + s * 1000, 1000)])


@functools.partial(
    pl.kernel,
    out_type=jax.ShapeDtypeStruct((NC, N, D), jnp.float32),
    mesh=_SC_MESH,
    scratch_types=[
        pltpu.VMEM((2, CH), jnp.int32),
        pltpu.VMEM((2, CH), jnp.int32),
        pltpu.VMEM((CH, D), jnp.float32),
        pltpu.VMEM((CH, D), jnp.float32),
        pltpu.VMEM((40, D), jnp.float32),
        pltpu.VMEM_SHARED((NACC, D), jnp.float32),
        pltpu.SemaphoreType.DMA,
        pltpu.SemaphoreType.DMA,
    ],
)
def _sc_aggregate(g_hbm, ef_hbm, zeros_hbm, out_hbm,
                  ibs0, ibs1, rows0_v, rows1_v, stage_v, acc_sh,
                  sem0, sem1):
    c = lax.axis_index("c")
    s = lax.axis_index("s")
    base, nch = _tile_chunks(c, s)

    pltpu.sync_copy(zeros_hbm, stage_v)
    for j in range(16):
        pltpu.sync_copy(stage_v, acc_sh.at[pl.ds(s * 640 + j * 40, 40)])

    pltpu.sync_copy(ef_hbm.at[:, pl.ds(base * CH, CH)], ibs0)
    pltpu.sync_copy(ef_hbm.at[:, pl.ds((base + 1) * CH, CH)], ibs1)
    plsc.subcore_barrier()

    pltpu.async_copy(g_hbm.at[ibs0.at[0]], rows0_v, sem0)

    def body(k, carry):
        i0 = 2 * k
        pltpu.async_copy(g_hbm.at[ibs1.at[0]], rows1_v, sem1)
        pltpu.make_async_copy(g_hbm.at[ibs0.at[0]], rows0_v, sem0).wait()
        pltpu.sync_copy(rows0_v, acc_sh.at[ibs0.at[1]], add=True)

        @pl.when(i0 + 2 < nch)
        def _next_even():
            pltpu.sync_copy(
                ef_hbm.at[:, pl.ds((base + i0 + 2) * CH, CH)], ibs0)
            pltpu.async_copy(g_hbm.at[ibs0.at[0]], rows0_v, sem0)

        pltpu.make_async_copy(g_hbm.at[ibs1.at[0]], rows1_v, sem1).wait()
        pltpu.sync_copy(rows1_v, acc_sh.at[ibs1.at[1]], add=True)

        @pl.when(i0 + 3 < nch)
        def _next_odd():
            pltpu.sync_copy(
                ef_hbm.at[:, pl.ds((base + i0 + 3) * CH, CH)], ibs1)

        return carry

    lax.fori_loop(0, nch // 2, body, 0, unroll=False)
    plsc.subcore_barrier()

    for j in range(16):
        blk = s + 16 * j

        @pl.when(blk < 250)
        def _wb():
            row = blk * 40
            pltpu.sync_copy(acc_sh.at[pl.ds(row, 40)], stage_v)
            pltpu.sync_copy(stage_v, out_hbm.at[c, pl.ds(row, 40)])


_BM = 2000


def _tc_scale_matmul_body(degA, degB, x_ref, w_ref, g_ref):
    dinv = lax.rsqrt(degA[...] + degB[...] + 1.0)
    h = jnp.dot(x_ref[...], w_ref[...], preferred_element_type=jnp.float32)
    g_ref[...] = h * dinv


def _tc_mid_body(degA, degB, aggA, aggB, g_ref, b_ref, w_ref, out_ref):
    dinv = lax.rsqrt(degA[...] + degB[...] + 1.0)
    h = (aggA[...] + aggB[...] + g_ref[...]) * dinv + b_ref[...]
    h = jnp.maximum(h, 0.0)
    out_ref[...] = jnp.dot(
        h, w_ref[...], preferred_element_type=jnp.float32) * dinv


def _tc_final_body(degA, degB, aggA, aggB, g_ref, b_ref, out_ref):
    dinv = lax.rsqrt(degA[...] + degB[...] + 1.0)
    h = (aggA[...] + aggB[...] + g_ref[...]) * dinv + b_ref[...]
    out_ref[...] = jnp.maximum(h, 0.0)


_col_spec = pl.BlockSpec((_BM, 1), lambda i: (i, 0))
_row_spec = pl.BlockSpec((_BM, D), lambda i: (i, 0))
_w_spec = pl.BlockSpec((D, D), lambda i: (0, 0))
_b_spec = pl.BlockSpec((1, D), lambda i: (0, 0))
_GRID = (N // _BM,)
_out_nd = jax.ShapeDtypeStruct((N, D), jnp.float32)

_tc_scale_matmul = pl.pallas_call(
    _tc_scale_matmul_body, grid=_GRID,
    in_specs=[_col_spec, _col_spec, _row_spec, _w_spec],
    out_specs=_row_spec, out_shape=_out_nd)

_tc_mid = pl.pallas_call(
    _tc_mid_body, grid=_GRID,
    in_specs=[_col_spec, _col_spec, _row_spec, _row_spec, _row_spec,
              _b_spec, _w_spec],
    out_specs=_row_spec, out_shape=_out_nd)

_tc_final = pl.pallas_call(
    _tc_final_body, grid=_GRID,
    in_specs=[_col_spec, _col_spec, _row_spec, _row_spec, _row_spec, _b_spec],
    out_specs=_row_spec, out_shape=_out_nd)


def kernel(x, edge_index, W1, b1, W2, b2):
    ef = edge_index
    zeros_n = jnp.zeros((1000,), jnp.float32)
    zeros_nd = jnp.zeros((40, D), jnp.float32)
    ones_ch = jnp.ones((CH,), jnp.float32)
    b1r = b1.reshape(1, D)
    b2r = b2.reshape(1, D)

    degp = _sc_degree(ef, ones_ch, zeros_n).reshape(NC, N)
    degA = degp[0][:, None]
    degB = degp[1][:, None]

    g1 = _tc_scale_matmul(degA, degB, x, W1)
    agg1 = _sc_aggregate(g1, ef, zeros_nd)
    g2 = _tc_mid(degA, degB, agg1[0], agg1[1], g1, b1r, W2)
    agg2 = _sc_aggregate(g2, ef, zeros_nd)
    out = _tc_final(degA, degB, agg2[0], agg2[1], g2, b2r)
    return out
